# Initial kernel scaffold; baseline (speedup 1.0000x reference)
import jax
import jax.numpy as jnp
from jax.experimental import pallas as pl


def kernel(height_positions, width_positions, pe_h, pe_w):
    raise NotImplementedError("write your pallas kernel here")


# speedup vs baseline: 10.3788x; 10.3788x over previous
import functools

import jax
import jax.numpy as jnp
from jax import lax
from jax.experimental import pallas as pl
from jax.experimental.pallas import tpu as pltpu
from jax.experimental.pallas import tpu_sc as plsc

_B = 1024
_N = 200
_HALF = 64
_NC = 2
_NS = 16
_NW = _NC * _NS
_E = (_B * _N) // _NW
_RPI = 128
_K = (2 * _E) // _RPI
_G = 4
_NGRP = _K // _G
_GROUP_ROWS = _G * _RPI


def _make_kernel():
  mesh = plsc.VectorSubcoreMesh(core_axis_name="c", subcore_axis_name="s")

  @functools.partial(
      pl.kernel,
      mesh=mesh,
      compiler_params=pltpu.CompilerParams(
          needs_layout_passes=False, use_tc_tiling_on_sc=False),
      out_type=jax.ShapeDtypeStruct((_B * _N * 2, _HALF), jnp.float32),
      scratch_types=[
          pltpu.VMEM((_E,), jnp.int32),
          pltpu.VMEM((_E,), jnp.int32),
          pltpu.VMEM((2 * _E,), jnp.int32),
          pltpu.VMEM((2, _GROUP_ROWS, _HALF), jnp.float32),
          pltpu.SemaphoreType.DMA,
          pltpu.SemaphoreType.DMA,
      ],
  )
  def body(h_hbm, w_hbm, tab_hbm, out_hbm, hbuf, wbuf, ibuf, gbuf, gsem, wsem):
    wid = lax.axis_index("s") * _NC + lax.axis_index("c")
    ebase = wid * _E
    pltpu.sync_copy(h_hbm.at[pl.ds(ebase, _E)], hbuf)
    pltpu.sync_copy(w_hbm.at[pl.ds(ebase, _E)], wbuf)

    def interleave(i, carry):
      hv = hbuf[pl.ds(i * 16, 16)]
      wv = wbuf[pl.ds(i * 16, 16)] + 1000
      flat = 32 * i + 2 * lax.iota(jnp.int32, 16)
      plsc.store_scatter(ibuf, [flat], hv)
      plsc.store_scatter(ibuf, [flat + 1], wv)
      return carry

    lax.fori_loop(0, _E // 16, interleave, 0)

    rbase = wid * 2 * _E
    write_handles = [None, None]
    for g in range(_NGRP):
      p = g & 1
      if write_handles[p] is not None:
        write_handles[p].wait()
      gather_handles = []
      for t in range(_G):
        j = g * _G + t
        gather_handles.append(
            pltpu.async_copy(
                tab_hbm.at[ibuf.at[pl.ds(j * _RPI, _RPI)]],
                gbuf.at[p, pl.ds(t * _RPI, _RPI)],
                gsem,
            )
        )
      for h in gather_handles:
        h.wait()
      write_handles[p] = pltpu.async_copy(
          gbuf.at[p],
          out_hbm.at[pl.ds(rbase + g * _GROUP_ROWS, _GROUP_ROWS)],
          wsem,
      )
    for h in write_handles:
      if h is not None:
        h.wait()

  return body


_gather_kernel = _make_kernel()


@jax.jit
def kernel(height_positions, width_positions, pe_h, pe_w):
  h = height_positions.reshape(-1)
  w = width_positions.reshape(-1)
  tab = jnp.concatenate([pe_h, pe_w], axis=0)
  out = _gather_kernel(h, w, tab)
  return out.reshape(_B, _N, 2 * _HALF)


# speedup vs baseline: 10.4425x; 1.0061x over previous
import functools

import jax
import jax.numpy as jnp
from jax import lax
from jax.experimental import pallas as pl
from jax.experimental.pallas import tpu as pltpu
from jax.experimental.pallas import tpu_sc as plsc

_B = 1024
_N = 200
_HALF = 64
_NC = 2
_NS = 16
_NW = _NC * _NS
_E = (_B * _N) // _NW
_RPI = 256
_K = (2 * _E) // _RPI
_G = 2
_NGRP = _K // _G
_GROUP_ROWS = _G * _RPI


def _make_kernel():
  mesh = plsc.VectorSubcoreMesh(core_axis_name="c", subcore_axis_name="s")

  @functools.partial(
      pl.kernel,
      mesh=mesh,
      compiler_params=pltpu.CompilerParams(
          needs_layout_passes=False, use_tc_tiling_on_sc=False),
      out_type=jax.ShapeDtypeStruct((_B * _N * 2, _HALF), jnp.float32),
      scratch_types=[
          pltpu.VMEM((_E,), jnp.int32),
          pltpu.VMEM((_E,), jnp.int32),
          pltpu.VMEM((2 * _E,), jnp.int32),
          pltpu.VMEM((2, _GROUP_ROWS, _HALF), jnp.float32),
          pltpu.SemaphoreType.DMA,
          pltpu.SemaphoreType.DMA,
      ],
  )
  def body(h_hbm, w_hbm, tab_hbm, out_hbm, hbuf, wbuf, ibuf, gbuf, gsem, wsem):
    wid = lax.axis_index("s") * _NC + lax.axis_index("c")
    ebase = wid * _E
    pltpu.sync_copy(h_hbm.at[pl.ds(ebase, _E)], hbuf)
    pltpu.sync_copy(w_hbm.at[pl.ds(ebase, _E)], wbuf)

    def interleave(i, carry):
      hv = hbuf[pl.ds(i * 16, 16)]
      wv = wbuf[pl.ds(i * 16, 16)] + 1000
      flat = 32 * i + 2 * lax.iota(jnp.int32, 16)
      plsc.store_scatter(ibuf, [flat], hv)
      plsc.store_scatter(ibuf, [flat + 1], wv)
      return carry

    lax.fori_loop(0, _E // 16, interleave, 0)

    rbase = wid * 2 * _E
    write_handles = [None, None]
    for g in range(_NGRP):
      p = g & 1
      if write_handles[p] is not None:
        write_handles[p].wait()
      gather_handles = []
      for t in range(_G):
        j = g * _G + t
        gather_handles.append(
            pltpu.async_copy(
                tab_hbm.at[ibuf.at[pl.ds(j * _RPI, _RPI)]],
                gbuf.at[p, pl.ds(t * _RPI, _RPI)],
                gsem,
            )
        )
      for h in gather_handles:
        h.wait()
      write_handles[p] = pltpu.async_copy(
          gbuf.at[p],
          out_hbm.at[pl.ds(rbase + g * _GROUP_ROWS, _GROUP_ROWS)],
          wsem,
      )
    for h in write_handles:
      if h is not None:
        h.wait()

  return body


_gather_kernel = _make_kernel()


@jax.jit
def kernel(height_positions, width_positions, pe_h, pe_w):
  h = height_positions.reshape(-1)
  w = width_positions.reshape(-1)
  tab = jnp.concatenate([pe_h, pe_w], axis=0)
  out = _gather_kernel(h, w, tab)
  return out.reshape(_B, _N, 2 * _HALF)


# speedup vs baseline: 21.3599x; 2.0455x over previous
import functools

import jax
import jax.numpy as jnp
from jax import lax
from jax.experimental import pallas as pl
from jax.experimental.pallas import tpu as pltpu
from jax.experimental.pallas import tpu_sc as plsc

_B = 1024
_N = 200
_HALF = 64
_NC = 2
_NS = 16
_NW = _NC * _NS
_E = (_B * _N) // _NW
_RPI = 256
_K = (2 * _E) // _RPI
_G = 2
_NGRP = _K // _G
_GROUP_ROWS = _G * _RPI


def _make_kernel():
  mesh = plsc.VectorSubcoreMesh(core_axis_name="c", subcore_axis_name="s")

  @functools.partial(
      pl.kernel,
      mesh=mesh,
      compiler_params=pltpu.CompilerParams(
          needs_layout_passes=False, use_tc_tiling_on_sc=False),
      out_type=jax.ShapeDtypeStruct((_B * _N * 2, _HALF), jnp.float32),
      scratch_types=[
          pltpu.VMEM((_E,), jnp.int32),
          pltpu.VMEM((_E,), jnp.int32),
          pltpu.VMEM((2 * _E,), jnp.int32),
          pltpu.VMEM((2, _GROUP_ROWS, _HALF), jnp.float32),
          pltpu.VMEM_SHARED((2000, _HALF), jnp.float32),
          pltpu.SemaphoreType.DMA,
          pltpu.SemaphoreType.DMA,
      ],
  )
  def body(h_hbm, w_hbm, tab_hbm, out_hbm, hbuf, wbuf, ibuf, gbuf, tab_sp,
           gsem, wsem):
    sid = lax.axis_index("s")
    wid = sid * _NC + lax.axis_index("c")
    ebase = wid * _E

    @pl.when(sid == 0)
    def _stage_table():
      pltpu.sync_copy(tab_hbm, tab_sp)

    pltpu.sync_copy(h_hbm.at[pl.ds(ebase, _E)], hbuf)
    pltpu.sync_copy(w_hbm.at[pl.ds(ebase, _E)], wbuf)

    def interleave(i, carry):
      hv = hbuf[pl.ds(i * 16, 16)]
      wv = wbuf[pl.ds(i * 16, 16)] + 1000
      flat = 32 * i + 2 * lax.iota(jnp.int32, 16)
      plsc.store_scatter(ibuf, [flat], hv)
      plsc.store_scatter(ibuf, [flat + 1], wv)
      return carry

    lax.fori_loop(0, _E // 16, interleave, 0)
    plsc.subcore_barrier()

    rbase = wid * 2 * _E
    write_handles = [None, None]
    for g in range(_NGRP):
      p = g & 1
      if write_handles[p] is not None:
        write_handles[p].wait()
      gather_handles = []
      for t in range(_G):
        j = g * _G + t
        gather_handles.append(
            pltpu.async_copy(
                tab_sp.at[ibuf.at[pl.ds(j * _RPI, _RPI)]],
                gbuf.at[p, pl.ds(t * _RPI, _RPI)],
                gsem,
            )
        )
      for h in gather_handles:
        h.wait()
      write_handles[p] = pltpu.async_copy(
          gbuf.at[p],
          out_hbm.at[pl.ds(rbase + g * _GROUP_ROWS, _GROUP_ROWS)],
          wsem,
      )
    for h in write_handles:
      if h is not None:
        h.wait()

  return body


_gather_kernel = _make_kernel()


@jax.jit
def kernel(height_positions, width_positions, pe_h, pe_w):
  h = height_positions.reshape(-1)
  w = width_positions.reshape(-1)
  tab = jnp.concatenate([pe_h, pe_w], axis=0)
  out = _gather_kernel(h, w, tab)
  return out.reshape(_B, _N, 2 * _HALF)
